# baseline (speedup 1.0000x reference)
import functools

import jax
import jax.numpy as jnp
from jax import lax
from jax.experimental import pallas as pl
from jax.experimental.pallas import tpu as pltpu
from jax.experimental.pallas import tpu_sc as plsc

NUM_SEGMENTS = 512
BLK = 128
NODES_BLK = BLK // 4
NW = 32
NBUF = 4
TC_ROWS = 4000
WINDOW = 64
N_SC = 16000


def _sc_body(nb_sc, base_n, x_hbm, idx_hbm, part_out, data_v,
             idx_all, zero_v, acc_sh, fetch_sems, scat_sem):
    core = lax.axis_index("c")
    sid = lax.axis_index("s")
    wid = core * 16 + sid

    zeros16 = jnp.zeros((16,), jnp.float32)
    for r in range(32):
        for c in range(8):
            zero_v[r, pl.ds(c * 16, 16)] = zeros16
    pltpu.sync_copy(zero_v, acc_sh.at[pl.ds(sid * 32, 32), :])
    plsc.subcore_barrier()

    extra = nb_sc - base_n * NW
    start = wid * base_n + jnp.minimum(wid, extra)
    has_extra = wid < extra

    qlo = start // 8
    ntiles = idx_all.shape[0]
    pltpu.sync_copy(idx_hbm.at[pl.ds(qlo, ntiles)], idx_all)

    def idx_row(b):
        g = start + b
        return idx_all.at[g // 8 - qlo, g % 8]

    def fetch(b, k):
        pltpu.async_copy(
            x_hbm.at[pl.ds((start + b) * BLK, BLK), :], data_v.at[k],
            fetch_sems.at[k],
        )

    def wait_fetch(b, k):
        pltpu.make_async_copy(
            x_hbm.at[pl.ds((start + b) * BLK, BLK), :], data_v.at[k],
            fetch_sems.at[k],
        ).wait()

    main_n = base_n - base_n % NBUF
    for k in range(NBUF):
        fetch(k, k)

    def ring(s, carry):
        for k in range(NBUF):
            b = s * NBUF + k
            wait_fetch(b, k)
            pltpu.async_copy(data_v.at[k], acc_sh.at[idx_row(b)], scat_sem,
                             add=True).wait()

            @pl.when(b + NBUF < main_n)
            def _():
                fetch(b + NBUF, k)

        return carry

    lax.fori_loop(0, main_n // NBUF, ring, 0)

    def tail_block(b):
        pltpu.sync_copy(x_hbm.at[pl.ds((start + b) * BLK, BLK), :],
                        data_v.at[0])
        pltpu.sync_copy(data_v.at[0], acc_sh.at[idx_row(b)], add=True)

    for b in range(main_n, base_n):
        tail_block(b)

    @pl.when(has_extra)
    def _():
        tail_block(base_n)

    plsc.subcore_barrier()

    pltpu.sync_copy(acc_sh.at[pl.ds(sid * 32, 32), :],
                    part_out.at[core, pl.ds(sid * 32, 32), :])


def _tc_accum(cmp, x_ref, acc_ref, base, width):
    oh = cmp.astype(jnp.bfloat16)
    m = jax.lax.dot(oh, x_ref[:, 0, :].astype(jnp.bfloat16),
                    preferred_element_type=jnp.float32)
    for s in range(1, 4):
        m += jax.lax.dot(oh, x_ref[:, s, :].astype(jnp.bfloat16),
                         preferred_element_type=jnp.float32)
    acc_ref[pl.ds(base, width), :] += m


def _tc_body(tc_off, idx_ref, x_ref, acc_out, cnt_out):
    i = pl.program_id(0)

    @pl.when(i == 0)
    def _():
        acc_out[...] = jnp.zeros_like(acc_out)
        cnt_out[...] = jnp.zeros_like(cnt_out)

    idx = idx_ref[0, 0, :]
    lo = jnp.min(idx)
    hi = jnp.max(idx)
    base = jnp.minimum((lo // 8) * 8, NUM_SEGMENTS - WINDOW)
    base = pl.multiple_of(base, 8)
    narrow = (hi - base) < WINDOW

    @pl.when(narrow)
    def _():
        seg = base + jax.lax.broadcasted_iota(
            jnp.int32, (WINDOW, idx.shape[0]), 0)
        cmp = seg == idx[None, :]
        cnt_out[pl.ds(base, WINDOW), :] += jnp.sum(
            cmp.astype(jnp.float32), axis=1, keepdims=True)

        @pl.when(i >= tc_off)
        def _():
            _tc_accum(cmp, x_ref, acc_out, base, WINDOW)

    @pl.when(jnp.logical_not(narrow))
    def _():
        seg = jax.lax.broadcasted_iota(
            jnp.int32, (NUM_SEGMENTS, idx.shape[0]), 0)
        cmp = seg == idx[None, :]
        cnt_out[...] += jnp.sum(cmp.astype(jnp.float32), axis=1,
                                keepdims=True)

        @pl.when(i >= tc_off)
        def _():
            _tc_accum(cmp, x_ref, acc_out, 0, NUM_SEGMENTS)


def _combine_body(p_ref, acc_ref, cnt_ref, out_ref):
    acc = p_ref[0] + p_ref[1] + acc_ref[...]
    out_ref[...] = acc / jnp.maximum(4.0 * cnt_ref[...], 4.0)


def kernel(node_distributions, batch_idx):
    n = node_distributions.shape[0]
    x4 = node_distributions.reshape(n * 4, 128)
    idx32 = batch_idx.astype(jnp.int32)

    nb_sc = N_SC * 4 // BLK
    base_n = nb_sc // NW
    mesh = plsc.VectorSubcoreMesh(core_axis_name="c", subcore_axis_name="s")
    sc_pool = pl.kernel(
        functools.partial(_sc_body, nb_sc, base_n),
        out_type=jax.ShapeDtypeStruct((2, NUM_SEGMENTS, 128), jnp.float32),
        mesh=mesh,
        scratch_types=[
            pltpu.VMEM((NBUF, BLK, 128), jnp.float32),
            pltpu.VMEM(((base_n + 8) // 8 + 1, 8, BLK), jnp.int32),
            pltpu.VMEM((32, 128), jnp.float32),
            pltpu.VMEM_SHARED((NUM_SEGMENTS, 128), jnp.float32),
            pltpu.SemaphoreType.DMA((NBUF,)),
            pltpu.SemaphoreType.DMA,
        ],
    )
    nt4 = -(-(n * 4) // (8 * BLK))
    idx4 = jnp.pad(jnp.repeat(idx32, 4), (0, nt4 * 8 * BLK - n * 4))
    partials = sc_pool(x4, idx4.reshape(nt4, 8, BLK))

    nblk = n // TC_ROWS
    tc_off = N_SC // TC_ROWS
    idx3 = idx32.reshape(nblk, 1, TC_ROWS)
    acc_tc, cnt_tc = pl.pallas_call(
        functools.partial(_tc_body, tc_off),
        grid=(nblk,),
        in_specs=[
            pl.BlockSpec((1, 1, TC_ROWS), lambda i: (i, 0, 0)),
            pl.BlockSpec((TC_ROWS, 4, 128),
                         lambda i: (jnp.maximum(i, tc_off), 0, 0)),
        ],
        out_specs=[
            pl.BlockSpec((NUM_SEGMENTS, 128), lambda i: (0, 0)),
            pl.BlockSpec((NUM_SEGMENTS, 1), lambda i: (0, 0)),
        ],
        out_shape=[
            jax.ShapeDtypeStruct((NUM_SEGMENTS, 128), jnp.float32),
            jax.ShapeDtypeStruct((NUM_SEGMENTS, 1), jnp.float32),
        ],
    )(idx3, node_distributions)

    out = pl.pallas_call(
        _combine_body,
        out_shape=jax.ShapeDtypeStruct((NUM_SEGMENTS, 128), jnp.float32),
    )(partials, acc_tc, cnt_tc)
    return out


# speedup vs baseline: 1.7931x; 1.7931x over previous
import functools

import jax
import jax.numpy as jnp
from jax import lax
from jax.experimental import pallas as pl
from jax.experimental.pallas import tpu as pltpu
from jax.experimental.pallas import tpu_sc as plsc

NUM_SEGMENTS = 512
BLK = 128
NODES_BLK = BLK // 4
NW = 32
NBUF = 4
TC_ROWS = 4000
WINDOW = 64
N_SC = 16000


def _sc_body(nb_sc, base_n, x_hbm, idx_hbm, part_out, data_v,
             idx_all, zero_v, acc_sh, fetch_sems, scat_sem):
    core = lax.axis_index("c")
    sid = lax.axis_index("s")
    wid = core * 16 + sid

    zeros16 = jnp.zeros((16,), jnp.float32)
    for r in range(32):
        for c in range(8):
            zero_v[r, pl.ds(c * 16, 16)] = zeros16
    pltpu.sync_copy(zero_v, acc_sh.at[pl.ds(sid * 32, 32), :])
    plsc.subcore_barrier()

    extra = nb_sc - base_n * NW
    start = wid * base_n + jnp.minimum(wid, extra)
    has_extra = wid < extra

    qlo = start // 8
    ntiles = idx_all.shape[0]
    pltpu.sync_copy(idx_hbm.at[pl.ds(qlo, ntiles)], idx_all)

    def idx_row(b):
        g = start + b
        return idx_all.at[g // 8 - qlo, g % 8]

    def fetch(b, k):
        pltpu.async_copy(
            x_hbm.at[pl.ds((start + b) * BLK, BLK), :], data_v.at[k],
            fetch_sems.at[k],
        )

    def wait_fetch(b, k):
        pltpu.make_async_copy(
            x_hbm.at[pl.ds((start + b) * BLK, BLK), :], data_v.at[k],
            fetch_sems.at[k],
        ).wait()

    main_n = base_n - base_n % NBUF
    for k in range(NBUF):
        fetch(k, k)

    def ring(s, carry):
        for k in range(NBUF):
            b = s * NBUF + k
            wait_fetch(b, k)
            pltpu.async_copy(data_v.at[k], acc_sh.at[idx_row(b)], scat_sem,
                             add=True).wait()

            @pl.when(b + NBUF < main_n)
            def _():
                fetch(b + NBUF, k)

        return carry

    lax.fori_loop(0, main_n // NBUF, ring, 0)

    def tail_block(b):
        pltpu.sync_copy(x_hbm.at[pl.ds((start + b) * BLK, BLK), :],
                        data_v.at[0])
        pltpu.sync_copy(data_v.at[0], acc_sh.at[idx_row(b)], add=True)

    for b in range(main_n, base_n):
        tail_block(b)

    @pl.when(has_extra)
    def _():
        tail_block(base_n)

    plsc.subcore_barrier()

    pltpu.sync_copy(acc_sh.at[pl.ds(sid * 32, 32), :],
                    part_out.at[core, pl.ds(sid * 32, 32), :])


def _tc_accum(cmp, do_data, x_ref, acc_ref, cnt_ref, base, width):
    cnt_ref[pl.ds(base, width), :] += jnp.sum(
        cmp.astype(jnp.float32), axis=1, keepdims=True)

    @pl.when(do_data)
    def _():
        oh = cmp.astype(jnp.bfloat16)
        m = jax.lax.dot(oh, x_ref[...].astype(jnp.bfloat16),
                        preferred_element_type=jnp.float32)
        acc_ref[pl.ds(base, width), :] += m


def _tc_body(tc_off, idx_ref, x_ref, acc_out, cnt_out):
    i = pl.program_id(0)

    @pl.when(i == 0)
    def _():
        acc_out[...] = jnp.zeros_like(acc_out)
        cnt_out[...] = jnp.zeros_like(cnt_out)

    idx4 = idx_ref[0, 0, :]
    lo = jnp.min(idx4)
    hi = jnp.max(idx4)
    base = jnp.minimum((lo // 8) * 8, NUM_SEGMENTS - WINDOW)
    base = pl.multiple_of(base, 8)
    narrow = (hi - base) < WINDOW
    do_data = i >= tc_off

    @pl.when(narrow)
    def _():
        seg = base + jax.lax.broadcasted_iota(
            jnp.int32, (WINDOW, idx4.shape[0]), 0)
        _tc_accum(seg == idx4[None, :], do_data, x_ref, acc_out, cnt_out,
                  base, WINDOW)

    @pl.when(jnp.logical_not(narrow))
    def _():
        seg = jax.lax.broadcasted_iota(
            jnp.int32, (NUM_SEGMENTS, idx4.shape[0]), 0)
        _tc_accum(seg == idx4[None, :], do_data, x_ref, acc_out, cnt_out,
                  0, NUM_SEGMENTS)


def _combine_body(p_ref, acc_ref, cnt_ref, out_ref):
    acc = p_ref[0] + p_ref[1] + acc_ref[...]
    out_ref[...] = acc / jnp.maximum(cnt_ref[...], 4.0)


def kernel(node_distributions, batch_idx):
    n = node_distributions.shape[0]
    x4 = node_distributions.reshape(n * 4, 128)
    idx32 = batch_idx.astype(jnp.int32)

    nb_sc = N_SC * 4 // BLK
    base_n = nb_sc // NW
    mesh = plsc.VectorSubcoreMesh(core_axis_name="c", subcore_axis_name="s")
    sc_pool = pl.kernel(
        functools.partial(_sc_body, nb_sc, base_n),
        out_type=jax.ShapeDtypeStruct((2, NUM_SEGMENTS, 128), jnp.float32),
        mesh=mesh,
        scratch_types=[
            pltpu.VMEM((NBUF, BLK, 128), jnp.float32),
            pltpu.VMEM(((base_n + 8) // 8 + 1, 8, BLK), jnp.int32),
            pltpu.VMEM((32, 128), jnp.float32),
            pltpu.VMEM_SHARED((NUM_SEGMENTS, 128), jnp.float32),
            pltpu.SemaphoreType.DMA((NBUF,)),
            pltpu.SemaphoreType.DMA,
        ],
    )
    nt4 = -(-(n * 4) // (8 * BLK))
    idx4 = jnp.repeat(idx32, 4)
    idx4_pad = jnp.pad(idx4, (0, nt4 * 8 * BLK - n * 4))
    partials = sc_pool(x4, idx4_pad.reshape(nt4, 8, BLK))

    nblk = n // TC_ROWS
    tc_off = N_SC // TC_ROWS
    idx43 = idx4.reshape(nblk, 1, TC_ROWS * 4)
    acc_tc, cnt_tc = pl.pallas_call(
        functools.partial(_tc_body, tc_off),
        grid=(nblk,),
        in_specs=[
            pl.BlockSpec((1, 1, TC_ROWS * 4), lambda i: (i, 0, 0)),
            pl.BlockSpec((TC_ROWS * 4, 128),
                         lambda i: (jnp.maximum(i, tc_off), 0)),
        ],
        out_specs=[
            pl.BlockSpec((NUM_SEGMENTS, 128), lambda i: (0, 0)),
            pl.BlockSpec((NUM_SEGMENTS, 1), lambda i: (0, 0)),
        ],
        out_shape=[
            jax.ShapeDtypeStruct((NUM_SEGMENTS, 128), jnp.float32),
            jax.ShapeDtypeStruct((NUM_SEGMENTS, 1), jnp.float32),
        ],
    )(idx43, x4)

    out = pl.pallas_call(
        _combine_body,
        out_shape=jax.ShapeDtypeStruct((NUM_SEGMENTS, 128), jnp.float32),
    )(partials, acc_tc, cnt_tc)
    return out


# speedup vs baseline: 3.1417x; 1.7522x over previous
import functools

import jax
import jax.numpy as jnp
from jax import lax
from jax.experimental import pallas as pl
from jax.experimental.pallas import tpu as pltpu
from jax.experimental.pallas import tpu_sc as plsc

NUM_SEGMENTS = 512
BLK = 128
NW = 32
NBUF = 4
TC_ROWS = 4000
WINDOW = 64
N_SC = 12000


def _sc_body(nb_sc, base_n, x_hbm, idx_hbm, part_out, data_v, idx_all, idx_x,
             zero_v, acc_sh, fetch_sems, scat_sem):
    core = lax.axis_index("c")
    sid = lax.axis_index("s")
    wid = core * 16 + sid

    zeros16 = jnp.zeros((16,), jnp.float32)
    for r in range(32):
        for c in range(8):
            zero_v[r, pl.ds(c * 16, 16)] = zeros16
    pltpu.sync_copy(zero_v, acc_sh.at[pl.ds(sid * 32, 32), :])
    plsc.subcore_barrier()

    extra = nb_sc - base_n * NW
    start = wid * base_n + jnp.minimum(wid, extra)
    has_extra = wid < extra

    pltpu.sync_copy(idx_hbm.at[pl.ds(start, base_n)], idx_all)

    def fetch(b, k):
        pltpu.async_copy(
            x_hbm.at[pl.ds((start + b) * BLK, BLK), :], data_v.at[k],
            fetch_sems.at[k],
        )

    def wait_fetch(b, k):
        pltpu.make_async_copy(
            x_hbm.at[pl.ds((start + b) * BLK, BLK), :], data_v.at[k],
            fetch_sems.at[k],
        ).wait()

    main_n = base_n - base_n % NBUF
    for k in range(NBUF):
        fetch(k, k)

    def ring(s, carry):
        for k in range(NBUF):
            b = s * NBUF + k
            wait_fetch(b, k)
            idx_row = idx_all.at[b, 0]
            pltpu.async_copy(data_v.at[k], acc_sh.at[idx_row], scat_sem,
                             add=True).wait()

            @pl.when(b + NBUF < main_n)
            def _():
                fetch(b + NBUF, k)

        return carry

    lax.fori_loop(0, main_n // NBUF, ring, 0)

    def tail_block(b, idx_row_ref):
        pltpu.sync_copy(x_hbm.at[pl.ds((start + b) * BLK, BLK), :],
                        data_v.at[0])
        pltpu.sync_copy(data_v.at[0], acc_sh.at[idx_row_ref], add=True)

    for b in range(main_n, base_n):
        tail_block(b, idx_all.at[b, 0])

    @pl.when(has_extra)
    def _():
        pltpu.sync_copy(idx_hbm.at[start + base_n], idx_x)
        tail_block(base_n, idx_x.at[0])

    plsc.subcore_barrier()

    pltpu.sync_copy(acc_sh.at[pl.ds(sid * 32, 32), :],
                    part_out.at[core, pl.ds(sid * 32, 32), :])


def _tc_accum(idx4, x2, acc_ref, cnt_ref, base, width):
    seg = base + jax.lax.broadcasted_iota(jnp.int32, (width, idx4.shape[0]), 0)
    cmp = seg == idx4[None, :]
    oh = cmp.astype(jnp.bfloat16)
    m = jax.lax.dot(oh, x2, preferred_element_type=jnp.float32)
    acc_ref[pl.ds(base, width), :] += m
    cnt_ref[pl.ds(base, width), :] += jnp.sum(
        cmp.astype(jnp.float32), axis=1, keepdims=True
    )


def _tc_body(idx_ref, x_ref, acc_out, cnt_out):
    i = pl.program_id(0)

    @pl.when(i == 0)
    def _():
        acc_out[...] = jnp.zeros_like(acc_out)
        cnt_out[...] = jnp.zeros_like(cnt_out)

    x2 = x_ref[...].astype(jnp.bfloat16)
    idx4 = idx_ref[0, 0, :]
    lo = jnp.min(idx4)
    hi = jnp.max(idx4)
    base = jnp.minimum((lo // 8) * 8, NUM_SEGMENTS - WINDOW)
    base = pl.multiple_of(base, 8)
    narrow = (hi - base) < WINDOW

    @pl.when(narrow)
    def _():
        _tc_accum(idx4, x2, acc_out, cnt_out, base, WINDOW)

    @pl.when(jnp.logical_not(narrow))
    def _():
        _tc_accum(idx4, x2, acc_out, cnt_out, 0, NUM_SEGMENTS)


def _combine_body(idx_ref, p_ref, acc_ref, cnt4_ref, out_ref, cnt_ref):
    i = pl.program_id(0)
    nblk = pl.num_programs(0)

    @pl.when(i == 0)
    def _():
        cnt_ref[...] = jnp.zeros_like(cnt_ref)

    idx = idx_ref[0, 0, :]
    seg = jax.lax.broadcasted_iota(jnp.int32, (NUM_SEGMENTS, TC_ROWS), 0)
    onehot = (seg == idx[None, :]).astype(jnp.float32)
    cnt_ref[...] += jnp.sum(onehot, axis=1, keepdims=True)

    @pl.when(i == nblk - 1)
    def _():
        acc = p_ref[0] + p_ref[1] + acc_ref[...]
        cnt4 = 4.0 * cnt_ref[...] + cnt4_ref[...]
        out_ref[...] = acc / jnp.maximum(cnt4, 4.0)


def kernel(node_distributions, batch_idx):
    n = node_distributions.shape[0]
    x4 = node_distributions.reshape(n * 4, 128)
    idx32 = batch_idx.astype(jnp.int32)
    idx4 = jnp.repeat(idx32, 4)

    nb_sc = N_SC * 4 // BLK
    base_n = nb_sc // NW
    idx4_sc = idx4.reshape(n * 4 // BLK, 1, BLK)
    mesh = plsc.VectorSubcoreMesh(core_axis_name="c", subcore_axis_name="s")
    sc_pool = pl.kernel(
        functools.partial(_sc_body, nb_sc, base_n),
        out_type=jax.ShapeDtypeStruct((2, NUM_SEGMENTS, 128), jnp.float32),
        mesh=mesh,
        scratch_types=[
            pltpu.VMEM((NBUF, BLK, 128), jnp.float32),
            pltpu.VMEM((base_n, 1, BLK), jnp.int32),
            pltpu.VMEM((1, BLK), jnp.int32),
            pltpu.VMEM((32, 128), jnp.float32),
            pltpu.VMEM_SHARED((NUM_SEGMENTS, 128), jnp.float32),
            pltpu.SemaphoreType.DMA((NBUF,)),
            pltpu.SemaphoreType.DMA,
        ],
    )
    partials = sc_pool(x4, idx4_sc)

    n_tc = n - N_SC
    nblk_tc = n_tc // TC_ROWS
    tc_off = N_SC // TC_ROWS
    idx4_tc = idx4.reshape(n // TC_ROWS, 1, TC_ROWS * 4)
    acc_tc, cnt4_tc = pl.pallas_call(
        _tc_body,
        grid=(nblk_tc,),
        in_specs=[
            pl.BlockSpec((1, 1, TC_ROWS * 4), lambda i: (i + tc_off, 0, 0)),
            pl.BlockSpec((TC_ROWS * 4, 128), lambda i: (i + tc_off, 0)),
        ],
        out_specs=[
            pl.BlockSpec((NUM_SEGMENTS, 128), lambda i: (0, 0)),
            pl.BlockSpec((NUM_SEGMENTS, 1), lambda i: (0, 0)),
        ],
        out_shape=[
            jax.ShapeDtypeStruct((NUM_SEGMENTS, 128), jnp.float32),
            jax.ShapeDtypeStruct((NUM_SEGMENTS, 1), jnp.float32),
        ],
    )(idx4_tc, x4)

    nblk_c = N_SC // TC_ROWS
    idx3 = idx32.reshape(n // TC_ROWS, 1, TC_ROWS)
    out = pl.pallas_call(
        _combine_body,
        grid=(nblk_c,),
        in_specs=[
            pl.BlockSpec((1, 1, TC_ROWS), lambda i: (i, 0, 0)),
            pl.BlockSpec((2, NUM_SEGMENTS, 128), lambda i: (0, 0, 0)),
            pl.BlockSpec((NUM_SEGMENTS, 128), lambda i: (0, 0)),
            pl.BlockSpec((NUM_SEGMENTS, 1), lambda i: (0, 0)),
        ],
        out_specs=pl.BlockSpec((NUM_SEGMENTS, 128), lambda i: (0, 0)),
        out_shape=jax.ShapeDtypeStruct((NUM_SEGMENTS, 128), jnp.float32),
        scratch_shapes=[pltpu.VMEM((NUM_SEGMENTS, 1), jnp.float32)],
    )(idx3, partials, acc_tc, cnt4_tc)
    return out
